# baseline (speedup 1.0000x reference)
import functools

import jax
import jax.numpy as jnp
from jax import lax
from jax.experimental import pallas as pl
from jax.experimental.pallas import tpu as pltpu
from jax.experimental.pallas import tpu_sc as plsc

_NC = 2
_NS = 16
_NW = _NC * _NS
_LANES = 16
_UNROLL = 8


def _make_sc_clip(n_total):
    per_w = n_total // _NW
    mesh = plsc.VectorSubcoreMesh(core_axis_name="c", subcore_axis_name="s")

    @functools.partial(
        pl.kernel,
        mesh=mesh,
        out_type=jax.ShapeDtypeStruct((n_total,), jnp.float32),
        scratch_types=[pltpu.VMEM((per_w,), jnp.float32)],
    )
    def sc_clip(w_hbm, out_hbm, w_v):
        wid = lax.axis_index("s") * _NC + lax.axis_index("c")
        base = wid * per_w
        pltpu.sync_copy(w_hbm.at[pl.ds(base, per_w)], w_v)

        chunk = _LANES * _UNROLL

        def step(i, carry):
            off = i * chunk
            for j in range(_UNROLL):
                x = w_v[pl.ds(off + j * _LANES, _LANES)]
                w_v[pl.ds(off + j * _LANES, _LANES)] = jnp.minimum(
                    jnp.maximum(x, 0.0), 1.0
                )
            return carry

        lax.fori_loop(0, per_w // chunk, step, 0)
        pltpu.sync_copy(w_v, out_hbm.at[pl.ds(base, per_w)])

    return sc_clip


def kernel(pre_spikes, post_spikes, weights):
    n_pre, n_post = weights.shape
    flat = weights.reshape(n_pre * n_post)
    out = _make_sc_clip(n_pre * n_post)(flat)
    return out.reshape(n_pre, n_post)


# speedup vs baseline: 4.4608x; 4.4608x over previous
import jax
import jax.numpy as jnp
from jax.experimental import pallas as pl

_BLOCK_ROWS = 128


def _clip_block(w_ref, o_ref):
    o_ref[...] = jnp.clip(w_ref[...], 0.0, 1.0)


def kernel(pre_spikes, post_spikes, weights):
    n_pre, n_post = weights.shape
    grid = (n_pre // _BLOCK_ROWS,)
    return pl.pallas_call(
        _clip_block,
        grid=grid,
        in_specs=[pl.BlockSpec((_BLOCK_ROWS, n_post), lambda i: (i, 0))],
        out_specs=pl.BlockSpec((_BLOCK_ROWS, n_post), lambda i: (i, 0)),
        out_shape=jax.ShapeDtypeStruct(weights.shape, weights.dtype),
    )(weights)


# speedup vs baseline: 8.1844x; 1.8347x over previous
import jax
import jax.numpy as jnp
from jax.experimental import pallas as pl

_BLOCK_ROWS = 512


def _clip_block(w_ref, o_ref):
    o_ref[...] = jnp.clip(w_ref[...], 0.0, 1.0)


def kernel(pre_spikes, post_spikes, weights):
    n_pre, n_post = weights.shape
    grid = (n_pre // _BLOCK_ROWS,)
    return pl.pallas_call(
        _clip_block,
        grid=grid,
        in_specs=[pl.BlockSpec((_BLOCK_ROWS, n_post), lambda i: (i, 0))],
        out_specs=pl.BlockSpec((_BLOCK_ROWS, n_post), lambda i: (i, 0)),
        out_shape=jax.ShapeDtypeStruct(weights.shape, weights.dtype),
    )(weights)
